# baseline (speedup 1.0000x reference)
import functools

import jax
import jax.numpy as jnp
from jax import lax
from jax.experimental import pallas as pl
from jax.experimental.pallas import tpu as pltpu
from jax.experimental.pallas import tpu_sc as plsc

BATCH = 16384
EMB = 32
NC = 2
NS = 16
NW = NC * NS
BPW = BATCH // NW
LANES = 16
GROUPS = BPW // LANES

_mesh = plsc.VectorSubcoreMesh(core_axis_name="c", subcore_axis_name="s")


@functools.partial(
    pl.kernel,
    mesh=_mesh,
    out_type=jax.ShapeDtypeStruct((BATCH,), jnp.float32),
    scratch_types=[
        pltpu.VMEM((BPW,), jnp.int32),
        pltpu.VMEM((BPW,), jnp.int32),
        pltpu.VMEM((BPW,), jnp.int32),
        pltpu.VMEM((BPW, EMB), jnp.float32),
        pltpu.VMEM((BPW, EMB), jnp.float32),
        pltpu.VMEM((BPW, EMB), jnp.float32),
        pltpu.VMEM((BPW,), jnp.float32),
        pltpu.SemaphoreType.DMA,
    ],
    compiler_params=pltpu.CompilerParams(
        needs_layout_passes=False, use_tc_tiling_on_sc=False
    ),
)
def _transe_sc(h_hbm, r_hbm, t_hbm, ent_hbm, rel_hbm, out_hbm,
               hi, ri, ti, hv, rv, tv, ov, sem):
    wid = lax.axis_index("s") * NC + lax.axis_index("c")
    base = wid * BPW

    pltpu.sync_copy(h_hbm.at[pl.ds(base, BPW)], hi)
    pltpu.sync_copy(r_hbm.at[pl.ds(base, BPW)], ri)
    pltpu.sync_copy(t_hbm.at[pl.ds(base, BPW)], ti)

    cp_h = pltpu.async_copy(ent_hbm.at[hi], hv, sem)
    cp_r = pltpu.async_copy(rel_hbm.at[ri], rv, sem)
    cp_t = pltpu.async_copy(ent_hbm.at[ti], tv, sem)
    cp_h.wait()
    cp_r.wait()
    cp_t.wait()

    lanes = lax.iota(jnp.int32, LANES)

    def group_body(g, carry):
        acc = jnp.zeros((LANES,), jnp.float32)
        for i in range(LANES):
            row = g * LANES + i
            h0 = hv[row, pl.ds(0, LANES)]
            h1 = hv[row, pl.ds(LANES, LANES)]
            r0 = rv[row, pl.ds(0, LANES)]
            r1 = rv[row, pl.ds(LANES, LANES)]
            t0 = tv[row, pl.ds(0, LANES)]
            t1 = tv[row, pl.ds(LANES, LANES)]
            e = jnp.abs(h0 + r0 - t0) + jnp.abs(h1 + r1 - t1)
            s = jnp.sum(e)
            acc = jnp.where(lanes == i, s, acc)
        ov[pl.ds(pl.multiple_of(g * LANES, LANES), LANES)] = acc
        return carry

    lax.fori_loop(0, GROUPS, group_body, 0)

    pltpu.sync_copy(ov, out_hbm.at[pl.ds(base, BPW)])


def kernel(batch_h, batch_r, batch_t, entity_embds, rel_embds):
    return _transe_sc(batch_h, batch_r, batch_t, entity_embds, rel_embds)
